# Initial kernel scaffold; baseline (speedup 1.0000x reference)
import jax
import jax.numpy as jnp
from jax.experimental import pallas as pl


def kernel(x, edge_attr, params, edge_index, batch_index):
    raise NotImplementedError("write your pallas kernel here")


# speedup vs baseline: 4.6341x; 4.6341x over previous
import functools

import jax
import jax.numpy as jnp
from jax import lax
from jax.experimental import pallas as pl
from jax.experimental.pallas import tpu as pltpu
from jax.experimental.pallas import tpu_sc as plsc

N = 10000
E = 320000
D = 128
G = 64
DN = 256

NC = 2
NS = 16
NW = NC * NS

CH = 128
EPT = ((E // NW) + CH - 1) // CH * CH
E_PAD = EPT * NW
NCHUNK = EPT // CH

RPT = 640
N_ACC = RPT * NS

RB = 32

_NEG_INF = float("-inf")


def _agg_body(feat, srcp, dstp, z2, z1, o1, acc_out, cnt_out,
              src_v, dst_v, rows_v, ones_v, zb_v, zc_v, acc, cnt, sem):
    cid = lax.axis_index("c")
    sid = lax.axis_index("s")
    gwid = cid * NS + sid

    pltpu.sync_copy(z2, zb_v)
    pltpu.sync_copy(z1, zc_v)
    pltpu.sync_copy(o1, ones_v)
    rs = sid * RPT
    for k in range(RPT // CH):
        pltpu.sync_copy(zb_v, acc.at[pl.ds(rs + k * CH, CH)])
    pltpu.sync_copy(zc_v, cnt.at[pl.ds(rs, RPT)])
    plsc.subcore_barrier()

    ebase = gwid * EPT

    def chunk(c, carry):
        off = ebase + c * CH
        pltpu.sync_copy(srcp.at[pl.ds(off, CH)], src_v)
        pltpu.sync_copy(dstp.at[pl.ds(off, CH)], dst_v)
        pltpu.async_copy(feat.at[src_v], rows_v, sem).wait()
        pltpu.sync_copy(rows_v, acc.at[dst_v], add=True)
        pltpu.sync_copy(ones_v, cnt.at[dst_v], add=True)
        return carry

    lax.fori_loop(0, NCHUNK, chunk, 0)
    plsc.subcore_barrier()

    pltpu.sync_copy(acc.at[pl.ds(rs, RPT)], acc_out.at[cid, pl.ds(rs, RPT)])
    pltpu.sync_copy(cnt.at[pl.ds(rs, RPT)], cnt_out.at[cid, pl.ds(rs, RPT)])


_agg = pl.kernel(
    _agg_body,
    out_type=(
        jax.ShapeDtypeStruct((NC, N_ACC, D), jnp.float32),
        jax.ShapeDtypeStruct((NC, N_ACC), jnp.float32),
    ),
    mesh=plsc.VectorSubcoreMesh(core_axis_name="c", subcore_axis_name="s"),
    scratch_types=[
        pltpu.VMEM((CH,), jnp.int32),
        pltpu.VMEM((CH,), jnp.int32),
        pltpu.VMEM((CH, D), jnp.float32),
        pltpu.VMEM((CH,), jnp.float32),
        pltpu.VMEM((CH, D), jnp.float32),
        pltpu.VMEM((RPT,), jnp.float32),
        pltpu.VMEM_SHARED((N_ACC, D), jnp.float32),
        pltpu.VMEM_SHARED((N_ACC,), jnp.float32),
        pltpu.SemaphoreType.DMA,
    ],
)


def _dense_a_body(acc_ref, cnt_ref, x_ref, wl_ref, bl_ref, wr_ref,
                  t_ref, st_ref):
    i = pl.program_id(0)
    s = acc_ref[0] + acc_ref[1]
    c = cnt_ref[0, 0] + cnt_ref[1, 0]
    cb = jnp.broadcast_to(c[:, None, :], (RPT // 128, 128, 128))
    cb = cb.reshape(RPT, 128)
    r_iota = lax.broadcasted_iota(jnp.int32, (RPT, 128), 0)
    l_iota = lax.broadcasted_iota(jnp.int32, (RPT, 128), 1)
    sel = (r_iota & 127) == l_iota
    ccol = jnp.sum(jnp.where(sel, cb, 0.0), axis=1, keepdims=True)
    aggr = s / jnp.maximum(ccol, 1.0)
    t = (jnp.dot(aggr, wl_ref[...], preferred_element_type=jnp.float32)
         + bl_ref[0]
         + jnp.dot(x_ref[...], wr_ref[...], preferred_element_type=jnp.float32))
    t_ref[...] = t
    rowid = lax.broadcasted_iota(jnp.int32, (RPT, D), 0) + i * RPT
    tm = jnp.where(rowid < N, t, 0.0)
    st = jnp.stack([jnp.sum(tm, axis=0), jnp.sum(tm * tm, axis=0)])

    @pl.when(i == 0)
    def _():
        st_ref[...] = st

    @pl.when(i > 0)
    def _():
        st_ref[...] += st


def _dense_a(acc_o, cnt3, h, wl, bl, wr):
    return pl.pallas_call(
        _dense_a_body,
        grid=(NS,),
        in_specs=[
            pl.BlockSpec((NC, RPT, D), lambda i: (0, i, 0)),
            pl.BlockSpec((NC, 1, RPT // 128, 128), lambda i: (0, i, 0, 0)),
            pl.BlockSpec((RPT, D), lambda i: (i, 0)),
            pl.BlockSpec((D, D), lambda i: (0, 0)),
            pl.BlockSpec((1, D), lambda i: (0, 0)),
            pl.BlockSpec((D, D), lambda i: (0, 0)),
        ],
        out_specs=[
            pl.BlockSpec((RPT, D), lambda i: (i, 0)),
            pl.BlockSpec((2, D), lambda i: (0, 0)),
        ],
        out_shape=[
            jax.ShapeDtypeStruct((N_ACC, D), jnp.float32),
            jax.ShapeDtypeStruct((2, D), jnp.float32),
        ],
    )(acc_o, cnt3, h, wl, bl, wr)


def _dense_b_body(t_ref, st_ref, g_ref, b_ref, h_ref):
    mu = st_ref[0] / N
    var = st_ref[1] / N - mu * mu
    inv = lax.rsqrt(var + 1e-5)
    h = (t_ref[...] - mu) * (g_ref[0] * inv) + b_ref[0]
    h_ref[...] = jnp.maximum(h, 0.0)


def _dense_b(t, st, gamma, beta):
    return pl.pallas_call(
        _dense_b_body,
        grid=(NS,),
        in_specs=[
            pl.BlockSpec((RPT, D), lambda i: (i, 0)),
            pl.BlockSpec((2, D), lambda i: (0, 0)),
            pl.BlockSpec((1, D), lambda i: (0, 0)),
            pl.BlockSpec((1, D), lambda i: (0, 0)),
        ],
        out_specs=pl.BlockSpec((RPT, D), lambda i: (i, 0)),
        out_shape=jax.ShapeDtypeStruct((N_ACC, D), jnp.float32),
    )(t, st, gamma, beta)


_GATHER_DNUMS = lax.GatherDimensionNumbers(
    offset_dims=(), collapsed_slice_dims=(0,), start_index_map=(0,))


def _lane_sum(v):
    iota = lax.iota(jnp.int32, 16)
    for sh in (1, 2, 4, 8):
        perm = jnp.bitwise_xor(iota, sh)
        v = v + lax.gather(v, perm[:, None], _GATHER_DNUMS, slice_sizes=(1,),
                           mode=lax.GatherScatterMode.PROMISE_IN_BOUNDS)
    return v[0]


def _pool_body(h, bip, gmax_out, gmean_out, bi_v, hbuf, omax_v, omean_v):
    cid = lax.axis_index("c")
    sid = lax.axis_index("s")
    gwid = cid * NS + sid
    g0 = gwid * 2

    pltpu.sync_copy(bip, bi_v)

    zero = jnp.zeros((16,), jnp.int32)

    one = jnp.ones((16,), jnp.int32)

    def pa(k, carry):
        c0, c1, c2 = carry
        ch = bi_v[pl.ds(k * 16, 16)]
        c0 = c0 + jnp.where(ch < g0, one, zero)
        c1 = c1 + jnp.where(ch < (g0 + 1), one, zero)
        c2 = c2 + jnp.where(ch < (g0 + 2), one, zero)
        return (c0, c1, c2)

    c0, c1, c2 = lax.fori_loop(0, N_ACC // 16, pa, (zero, zero, zero))
    b0 = _lane_sum(c0)
    b1 = _lane_sum(c1)
    b2 = _lane_sum(c2)

    for gi in range(2):
        s = jnp.where(gi == 0, b0, b1)
        e = jnp.where(gi == 0, b1, b2)
        n = e - s
        base0 = pl.multiple_of(lax.shift_left(
            lax.shift_right_logical(s, 5), 5), RB)
        nch = jnp.where(
            n > 0, lax.shift_right_logical(e - base0 + (RB - 1), 5), 0)

        init = tuple([jnp.full((16,), _NEG_INF, jnp.float32)] * 8
                     + [jnp.zeros((16,), jnp.float32)] * 8)

        def chunk(k, carry, s=s, e=e, base0=base0):
            start = pl.multiple_of(base0 + k * RB, RB)
            pltpu.sync_copy(h.at[pl.ds(start, RB)], hbuf)
            vals = list(carry)
            for r in range(RB):
                ridx = start + r
                valid = jnp.logical_and(ridx >= s, ridx < e)
                for j in range(8):
                    row = hbuf[r, pl.ds(j * 16, 16)]
                    vals[j] = jnp.maximum(
                        vals[j], jnp.where(valid, row, _NEG_INF))
                    vals[8 + j] = vals[8 + j] + jnp.where(valid, row, 0.0)
            return tuple(vals)

        res = lax.fori_loop(0, nch, chunk, init)
        nv = jnp.full((16,), 1.0, jnp.float32) * n.astype(jnp.float32)
        rn = jnp.full((16,), 1.0, jnp.float32) / jnp.maximum(nv, 1.0)
        for j in range(8):
            omax_v[gi, pl.ds(j * 16, 16)] = res[j]
            omean_v[gi, pl.ds(j * 16, 16)] = res[8 + j] * rn
    pltpu.sync_copy(omax_v, gmax_out.at[gwid])
    pltpu.sync_copy(omean_v, gmean_out.at[gwid])


_pool = pl.kernel(
    _pool_body,
    out_type=(
        jax.ShapeDtypeStruct((NW, 2, D), jnp.float32),
        jax.ShapeDtypeStruct((NW, 2, D), jnp.float32),
    ),
    mesh=plsc.VectorSubcoreMesh(core_axis_name="c", subcore_axis_name="s"),
    scratch_types=[
        pltpu.VMEM((N_ACC,), jnp.int32),
        pltpu.VMEM((RB, D), jnp.float32),
        pltpu.VMEM((2, D), jnp.float32),
        pltpu.VMEM((2, D), jnp.float32),
    ],
)


def _mlp_body(gmax_ref, gmean_ref, w1_ref, b1_ref, w2_ref, b2_ref,
              w3_ref, b3_ref, o_ref):
    z = jnp.concatenate([gmax_ref[...], gmean_ref[...]], axis=1)
    z = jnp.maximum(
        jnp.dot(z, w1_ref[...], preferred_element_type=jnp.float32)
        + b1_ref[0], 0.0)
    z = jnp.maximum(
        jnp.dot(z, w2_ref[...], preferred_element_type=jnp.float32)
        + b2_ref[0], 0.0)
    o_ref[...] = (jnp.dot(z, w3_ref[...], preferred_element_type=jnp.float32)
                  + b3_ref[0])


def _mlp(gmax, gmean, w1, b1, w2, b2, w3p, b3p):
    return pl.pallas_call(
        _mlp_body,
        out_shape=jax.ShapeDtypeStruct((G, 128), jnp.float32),
    )(gmax, gmean, w1, b1, w2, b2, w3p, b3p)


def kernel(x, edge_attr, params, edge_index, batch_index):
    del edge_attr
    f32 = jnp.float32
    xp = jnp.zeros((N_ACC, D), f32).at[:N].set(x)
    pad = E_PAD - E
    srcp = jnp.concatenate([edge_index[0], jnp.zeros((pad,), jnp.int32)])
    dstp = jnp.concatenate([edge_index[1], jnp.full((pad,), N, jnp.int32)])
    z2 = jnp.zeros((CH, D), f32)
    z1 = jnp.zeros((RPT,), f32)
    o1 = jnp.ones((CH,), f32)
    bip = jnp.concatenate(
        [batch_index, jnp.full((N_ACC - N,), 1 << 20, jnp.int32)])

    h = xp
    for i in range(2):
        p = params[f"conv{i}"]
        bn = params[f"bn{i}"]
        acc_o, cnt_o = _agg(h, srcp, dstp, z2, z1, o1)
        cnt3 = cnt_o.reshape(NC, NS, RPT // 128, 128)
        t, st = _dense_a(acc_o, cnt3, h, p["Wl"], p["bl"].reshape(1, D),
                         p["Wr"])
        h = _dense_b(t, st, bn["gamma"].reshape(1, D),
                     bn["beta"].reshape(1, D))

    gmax, gmean = _pool(h, bip)
    gmax = gmax.reshape(G, D)
    gmean = gmean.reshape(G, D)

    w3p = jnp.pad(params["W3"], ((0, 0), (0, 127)))
    b3p = jnp.broadcast_to(params["b3"].reshape(1, 1), (1, 128))
    out = _mlp(gmax, gmean, params["W1"], params["b1"].reshape(1, DN),
               params["W2"], params["b2"].reshape(1, DN // 2), w3p, b3p)
    return out[:, :1]
